# baseline (speedup 1.0000x reference)
import functools

import jax
import jax.numpy as jnp
from jax import lax
from jax.experimental import pallas as pl
from jax.experimental.pallas import tpu as pltpu
from jax.experimental.pallas import tpu_sc as plsc

N = 10000
E = 320000
D = 128
OUT = 10
B = 2000

NC = 2
NS = 16
NW = NC * NS
EPW = E // NW
CH = 50
NCHUNK = EPW // CH
NHALF = 5
NCH = NCHUNK // NHALF
NBANK = 4
RPT = 624
RTAIL = N - NS * RPT

@functools.cache
def _make_sc_aggregate():
    mesh = plsc.VectorSubcoreMesh(core_axis_name="c", subcore_axis_name="s")

    @functools.partial(
        pl.kernel,
        mesh=mesh,
        out_type=jax.ShapeDtypeStruct((NC, N, D), jnp.float32),
        scratch_types=[
            pltpu.VMEM((NCH, CH), jnp.int32),
            pltpu.VMEM((NCH, CH), jnp.int32),
            pltpu.VMEM((NBANK, CH, D), jnp.float32),
            pltpu.VMEM_SHARED((N, D), jnp.float32),
            pltpu.SemaphoreType.DMA,
            pltpu.SemaphoreType.DMA,
            pltpu.SemaphoreType.DMA,
            pltpu.SemaphoreType.DMA,
        ],
    )
    def _sc_aggregate(h_hbm, srcr_hbm, dstr_hbm, out_hbm, sidx, didx, rows,
                      acc, gsem_a, gsem_b, gsem_c, gsem_d):
        c = lax.axis_index("c")
        s = lax.axis_index("s")
        wid = s * NC + c
        r0 = s * RPT
        pltpu.sync_copy(h_hbm.at[pl.ds(r0, RPT)], acc.at[pl.ds(r0, RPT)])

        @pl.when(s == NS - 1)
        def _():
            pltpu.sync_copy(
                h_hbm.at[pl.ds(NS * RPT, RTAIL)], acc.at[pl.ds(NS * RPT, RTAIL)]
            )

        plsc.subcore_barrier()

        sems = (gsem_a, gsem_b, gsem_c, gsem_d)

        def fire(j, bank):
            pltpu.async_copy(h_hbm.at[sidx.at[j]], rows.at[bank], sems[bank])

        def drain_scatter(j, bank):
            pltpu.make_async_copy(
                h_hbm.at[sidx.at[0]], rows.at[bank], sems[bank]
            ).wait()
            pltpu.sync_copy(rows.at[bank], acc.at[didx.at[j]], add=True)

        for hf in range(NHALF):
            pltpu.sync_copy(srcr_hbm.at[wid, hf], sidx)
            pltpu.sync_copy(dstr_hbm.at[wid, hf], didx)
            for k in range(NBANK - 1):
                fire(k, k)

            def body(i, carry):
                for k in range(NBANK):
                    j = i * NBANK + k
                    drain_scatter(j, k)

                    @pl.when(j + NBANK - 1 < NCH)
                    def _():
                        fire(j + NBANK - 1, (k + NBANK - 1) % NBANK)

                return carry

            lax.fori_loop(0, NCH // NBANK, body, 0)
        plsc.subcore_barrier()
        pltpu.sync_copy(acc.at[pl.ds(r0, RPT)], out_hbm.at[c, pl.ds(r0, RPT)])

        @pl.when(s == NS - 1)
        def _():
            pltpu.sync_copy(
                acc.at[pl.ds(NS * RPT, RTAIL)],
                out_hbm.at[c, pl.ds(NS * RPT, RTAIL)],
            )

    return _sc_aggregate


def _mlp_body(h_ref, p0_ref, p1_ref, wa_ref, ba_ref, wb_ref, bb_ref, o_ref):
    m = p0_ref[...] + p1_ref[...] - h_ref[...]
    t = jnp.maximum(
        jnp.dot(m, wa_ref[...], preferred_element_type=jnp.float32) + ba_ref[...],
        0.0,
    )
    o_ref[...] = jnp.maximum(
        jnp.dot(t, wb_ref[...], preferred_element_type=jnp.float32) + bb_ref[...],
        0.0,
    )


_ROWBLK = 2000


def _mlp(h, p0, p1, wa, ba, wb, bb):
    return pl.pallas_call(
        _mlp_body,
        grid=(N // _ROWBLK,),
        in_specs=[
            pl.BlockSpec((_ROWBLK, D), lambda i: (i, 0)),
            pl.BlockSpec((_ROWBLK, D), lambda i: (i, 0)),
            pl.BlockSpec((_ROWBLK, D), lambda i: (i, 0)),
            pl.BlockSpec((D, D), lambda i: (0, 0)),
            pl.BlockSpec((1, D), lambda i: (0, 0)),
            pl.BlockSpec((D, D), lambda i: (0, 0)),
            pl.BlockSpec((1, D), lambda i: (0, 0)),
        ],
        out_specs=pl.BlockSpec((_ROWBLK, D), lambda i: (i, 0)),
        out_shape=jax.ShapeDtypeStruct((N, D), jnp.float32),
    )(h, p0, p1, wa, ba.reshape(1, D), wb, bb.reshape(1, D))


def _fc_body(x_ref, w_ref, b_ref, o_ref):
    o_ref[...] = (
        jnp.dot(x_ref[...], w_ref[...], preferred_element_type=jnp.float32)
        + b_ref[...]
    )


def _fc(x, w, b):
    return pl.pallas_call(
        _fc_body,
        out_shape=jax.ShapeDtypeStruct((B, OUT), jnp.float32),
    )(x, w, b.reshape(1, OUT))


def kernel(x, edge_index, W0a, b0a, W0b, b0b, W1a, b1a, W1b, b1b, W2a, b2a,
           W2b, b2b, W3a, b3a, W3b, b3b, W4a, b4a, W4b, b4b, Wfc, bfc):
    src = edge_index[0]
    dst = edge_index[1]
    layers = [
        (W0a, b0a, W0b, b0b),
        (W1a, b1a, W1b, b1b),
        (W2a, b2a, W2b, b2b),
        (W3a, b3a, W3b, b3b),
        (W4a, b4a, W4b, b4b),
    ]
    h = x
    sc_aggregate = _make_sc_aggregate()
    srcr = src.reshape(NW, NHALF, NCH, CH)
    dstr = dst.reshape(NW, NHALF, NCH, CH)
    for (Wa, ba, Wb, bb) in layers:
        parts = sc_aggregate(h, srcr, dstr)
        h = _mlp(h, parts[0], parts[1], Wa, ba, Wb, bb)
    return _fc(h.reshape(B, -1), Wfc, bfc)


# speedup vs baseline: 1.0616x; 1.0616x over previous
import functools

import jax
import jax.numpy as jnp
from jax import lax
from jax.experimental import pallas as pl
from jax.experimental.pallas import tpu as pltpu
from jax.experimental.pallas import tpu_sc as plsc

N = 10000
E = 320000
D = 128
OUT = 10
B = 2000

NC = 2
NS = 16
NW = NC * NS
EPW = E // NW
CH = 125
NCHUNK = EPW // CH
NHALF = 5
NCH = NCHUNK // NHALF
RPT = 624
RTAIL = N - NS * RPT

@functools.cache
def _make_sc_aggregate():
    mesh = plsc.VectorSubcoreMesh(core_axis_name="c", subcore_axis_name="s")

    @functools.partial(
        pl.kernel,
        mesh=mesh,
        out_type=jax.ShapeDtypeStruct((NC, N, D), jnp.float32),
        scratch_types=[
            pltpu.VMEM((2, NCH, CH), jnp.int32),
            pltpu.VMEM((2, NCH, CH), jnp.int32),
            pltpu.VMEM((2, CH, D), jnp.float32),
            pltpu.VMEM_SHARED((N, D), jnp.float32),
            pltpu.SemaphoreType.DMA,
            pltpu.SemaphoreType.DMA,
            pltpu.SemaphoreType.DMA,
            pltpu.SemaphoreType.DMA,
        ],
    )
    def _sc_aggregate(h_hbm, srcr_hbm, dstr_hbm, out_hbm, sidx, didx, rows,
                      acc, gsem_a, gsem_b, isem, csem):
        c = lax.axis_index("c")
        s = lax.axis_index("s")
        wid = s * NC + c
        r0 = s * RPT
        pltpu.async_copy(h_hbm.at[pl.ds(r0, RPT)], acc.at[pl.ds(r0, RPT)], csem)

        @pl.when(s == NS - 1)
        def _():
            pltpu.async_copy(
                h_hbm.at[pl.ds(NS * RPT, RTAIL)], acc.at[pl.ds(NS * RPT, RTAIL)],
                csem,
            )

        sems = (gsem_a, gsem_b)

        def stage(hf, qb, do_wait):
            cp_s = pltpu.async_copy(srcr_hbm.at[wid, hf], sidx.at[qb], isem)
            cp_d = pltpu.async_copy(dstr_hbm.at[wid, hf], didx.at[qb], isem)
            if do_wait:
                cp_s.wait()
                cp_d.wait()

        def fire(qb, j, bank):
            pltpu.async_copy(h_hbm.at[sidx.at[qb, j]], rows.at[bank], sems[bank])

        def drain_scatter(qb, j, bank):
            pltpu.make_async_copy(
                h_hbm.at[sidx.at[0, 0]], rows.at[bank], sems[bank]
            ).wait()
            pltpu.sync_copy(rows.at[bank], acc.at[didx.at[qb, j]], add=True)

        stage(0, 0, True)
        fire(0, 0, 0)

        pltpu.make_async_copy(
            h_hbm.at[pl.ds(r0, RPT)], acc.at[pl.ds(r0, RPT)], csem
        ).wait()

        @pl.when(s == NS - 1)
        def _():
            pltpu.make_async_copy(
                h_hbm.at[pl.ds(NS * RPT, RTAIL)], acc.at[pl.ds(NS * RPT, RTAIL)],
                csem,
            ).wait()

        plsc.subcore_barrier()

        for hf in range(NHALF):
            qb = hf % 2
            if hf + 1 < NHALF:
                stage(hf + 1, 1 - qb, False)

            def body(i, carry, qb=qb):
                j0 = i * 2
                fire(qb, j0 + 1, 1)
                drain_scatter(qb, j0, 0)

                @pl.when(j0 + 2 < NCH)
                def _():
                    fire(qb, j0 + 2, 0)

                drain_scatter(qb, j0 + 1, 1)
                return carry

            lax.fori_loop(0, NCH // 2, body, 0)

            if hf + 1 < NHALF:
                pltpu.make_async_copy(
                    srcr_hbm.at[wid, hf + 1], sidx.at[1 - qb], isem
                ).wait()
                pltpu.make_async_copy(
                    dstr_hbm.at[wid, hf + 1], didx.at[1 - qb], isem
                ).wait()
                fire(1 - qb, 0, 0)
        plsc.subcore_barrier()
        pltpu.sync_copy(acc.at[pl.ds(r0, RPT)], out_hbm.at[c, pl.ds(r0, RPT)])

        @pl.when(s == NS - 1)
        def _():
            pltpu.sync_copy(
                acc.at[pl.ds(NS * RPT, RTAIL)],
                out_hbm.at[c, pl.ds(NS * RPT, RTAIL)],
            )

    return _sc_aggregate


def _mlp_body(h_ref, p0_ref, p1_ref, wa_ref, ba_ref, wb_ref, bb_ref, o_ref):
    m = p0_ref[...] + p1_ref[...] - h_ref[...]
    t = jnp.maximum(
        jnp.dot(m, wa_ref[...], preferred_element_type=jnp.float32) + ba_ref[...],
        0.0,
    )
    o_ref[...] = jnp.maximum(
        jnp.dot(t, wb_ref[...], preferred_element_type=jnp.float32) + bb_ref[...],
        0.0,
    )


_ROWBLK = 2000


def _mlp(h, p0, p1, wa, ba, wb, bb):
    return pl.pallas_call(
        _mlp_body,
        grid=(N // _ROWBLK,),
        in_specs=[
            pl.BlockSpec((_ROWBLK, D), lambda i: (i, 0)),
            pl.BlockSpec((_ROWBLK, D), lambda i: (i, 0)),
            pl.BlockSpec((_ROWBLK, D), lambda i: (i, 0)),
            pl.BlockSpec((D, D), lambda i: (0, 0)),
            pl.BlockSpec((1, D), lambda i: (0, 0)),
            pl.BlockSpec((D, D), lambda i: (0, 0)),
            pl.BlockSpec((1, D), lambda i: (0, 0)),
        ],
        out_specs=pl.BlockSpec((_ROWBLK, D), lambda i: (i, 0)),
        out_shape=jax.ShapeDtypeStruct((N, D), jnp.float32),
    )(h, p0, p1, wa, ba.reshape(1, D), wb, bb.reshape(1, D))


def _fc_body(x_ref, w_ref, b_ref, o_ref):
    o_ref[...] = (
        jnp.dot(x_ref[...], w_ref[...], preferred_element_type=jnp.float32)
        + b_ref[...]
    )


def _fc(x, w, b):
    return pl.pallas_call(
        _fc_body,
        out_shape=jax.ShapeDtypeStruct((B, OUT), jnp.float32),
    )(x, w, b.reshape(1, OUT))


def kernel(x, edge_index, W0a, b0a, W0b, b0b, W1a, b1a, W1b, b1b, W2a, b2a,
           W2b, b2b, W3a, b3a, W3b, b3b, W4a, b4a, W4b, b4b, Wfc, bfc):
    src = edge_index[0]
    dst = edge_index[1]
    layers = [
        (W0a, b0a, W0b, b0b),
        (W1a, b1a, W1b, b1b),
        (W2a, b2a, W2b, b2b),
        (W3a, b3a, W3b, b3b),
        (W4a, b4a, W4b, b4b),
    ]
    h = x
    sc_aggregate = _make_sc_aggregate()
    srcr = src.reshape(NW, NHALF, NCH, CH)
    dstr = dst.reshape(NW, NHALF, NCH, CH)
    for (Wa, ba, Wb, bb) in layers:
        parts = sc_aggregate(h, srcr, dstr)
        h = _mlp(h, parts[0], parts[1], Wa, ba, Wb, bb)
    return _fc(h.reshape(B, -1), Wfc, bfc)


# speedup vs baseline: 1.0744x; 1.0121x over previous
import functools

import jax
import jax.numpy as jnp
from jax import lax
from jax.experimental import pallas as pl
from jax.experimental.pallas import tpu as pltpu
from jax.experimental.pallas import tpu_sc as plsc

N = 10000
E = 320000
D = 128
OUT = 10
B = 2000

NC = 2
NS = 16
NW = NC * NS
EPW = E // NW
CH = 125
NCHUNK = EPW // CH
NHALF = 5
NCH = NCHUNK // NHALF
RPT = 624
RTAIL = N - NS * RPT

@functools.cache
def _make_sc_aggregate():
    mesh = plsc.VectorSubcoreMesh(core_axis_name="c", subcore_axis_name="s")

    @functools.partial(
        pl.kernel,
        mesh=mesh,
        out_type=jax.ShapeDtypeStruct((NC, N, D), jnp.float32),
        scratch_types=[
            pltpu.VMEM((2, NCH, CH), jnp.int32),
            pltpu.VMEM((2, NCH, CH), jnp.int32),
            pltpu.VMEM((2, CH, D), jnp.float32),
            pltpu.VMEM_SHARED((N, D), jnp.float32),
            pltpu.SemaphoreType.DMA,
            pltpu.SemaphoreType.DMA,
            pltpu.SemaphoreType.DMA,
            pltpu.SemaphoreType.DMA,
        ],
    )
    def _sc_aggregate(h_hbm, z_hbm, srcr_hbm, dstr_hbm, out_hbm, sidx, didx,
                      rows, acc, gsem_a, gsem_b, isem, csem):
        c = lax.axis_index("c")
        s = lax.axis_index("s")
        wid = s * NC + c
        r0 = s * RPT
        init = (h_hbm, z_hbm)
        for cc in range(NC):

            @pl.when(c == cc)
            def _(cc=cc):
                pltpu.async_copy(
                    init[cc].at[pl.ds(r0, RPT)], acc.at[pl.ds(r0, RPT)], csem
                )

                @pl.when(s == NS - 1)
                def _():
                    pltpu.async_copy(
                        init[cc].at[pl.ds(NS * RPT, RTAIL)],
                        acc.at[pl.ds(NS * RPT, RTAIL)],
                        csem,
                    )

        sems = (gsem_a, gsem_b)

        def stage(hf, qb, do_wait):
            cp_s = pltpu.async_copy(srcr_hbm.at[wid, hf], sidx.at[qb], isem)
            cp_d = pltpu.async_copy(dstr_hbm.at[wid, hf], didx.at[qb], isem)
            if do_wait:
                cp_s.wait()
                cp_d.wait()

        def fire(qb, j, bank):
            pltpu.async_copy(h_hbm.at[sidx.at[qb, j]], rows.at[bank], sems[bank])

        def drain_scatter(qb, j, bank):
            pltpu.make_async_copy(
                h_hbm.at[sidx.at[0, 0]], rows.at[bank], sems[bank]
            ).wait()
            pltpu.sync_copy(rows.at[bank], acc.at[didx.at[qb, j]], add=True)

        stage(0, 0, True)
        fire(0, 0, 0)

        pltpu.make_async_copy(
            h_hbm.at[pl.ds(r0, RPT)], acc.at[pl.ds(r0, RPT)], csem
        ).wait()

        @pl.when(s == NS - 1)
        def _():
            pltpu.make_async_copy(
                h_hbm.at[pl.ds(NS * RPT, RTAIL)], acc.at[pl.ds(NS * RPT, RTAIL)],
                csem,
            ).wait()

        plsc.subcore_barrier()

        for hf in range(NHALF):
            qb = hf % 2
            if hf + 1 < NHALF:
                stage(hf + 1, 1 - qb, False)

            def body(i, carry, qb=qb):
                j0 = i * 2
                fire(qb, j0 + 1, 1)
                drain_scatter(qb, j0, 0)

                @pl.when(j0 + 2 < NCH)
                def _():
                    fire(qb, j0 + 2, 0)

                drain_scatter(qb, j0 + 1, 1)
                return carry

            lax.fori_loop(0, NCH // 2, body, 0)

            if hf + 1 < NHALF:
                pltpu.make_async_copy(
                    srcr_hbm.at[wid, hf + 1], sidx.at[1 - qb], isem
                ).wait()
                pltpu.make_async_copy(
                    dstr_hbm.at[wid, hf + 1], didx.at[1 - qb], isem
                ).wait()
                fire(1 - qb, 0, 0)
        plsc.subcore_barrier()
        pltpu.sync_copy(acc.at[pl.ds(r0, RPT)], out_hbm.at[c, pl.ds(r0, RPT)])

        @pl.when(s == NS - 1)
        def _():
            pltpu.sync_copy(
                acc.at[pl.ds(NS * RPT, RTAIL)],
                out_hbm.at[c, pl.ds(NS * RPT, RTAIL)],
            )

    return _sc_aggregate


def _mlp_body(p0_ref, p1_ref, wa_ref, ba_ref, wb_ref, bb_ref, o_ref):
    m = p0_ref[...] + p1_ref[...]
    t = jnp.maximum(
        jnp.dot(m, wa_ref[...], preferred_element_type=jnp.float32) + ba_ref[...],
        0.0,
    )
    o_ref[...] = jnp.maximum(
        jnp.dot(t, wb_ref[...], preferred_element_type=jnp.float32) + bb_ref[...],
        0.0,
    )


_ROWBLK = 2000


def _mlp(p0, p1, wa, ba, wb, bb):
    return pl.pallas_call(
        _mlp_body,
        grid=(N // _ROWBLK,),
        in_specs=[
            pl.BlockSpec((_ROWBLK, D), lambda i: (i, 0)),
            pl.BlockSpec((_ROWBLK, D), lambda i: (i, 0)),
            pl.BlockSpec((D, D), lambda i: (0, 0)),
            pl.BlockSpec((1, D), lambda i: (0, 0)),
            pl.BlockSpec((D, D), lambda i: (0, 0)),
            pl.BlockSpec((1, D), lambda i: (0, 0)),
        ],
        out_specs=pl.BlockSpec((_ROWBLK, D), lambda i: (i, 0)),
        out_shape=jax.ShapeDtypeStruct((N, D), jnp.float32),
    )(p0, p1, wa, ba.reshape(1, D), wb, bb.reshape(1, D))


def _fc_body(x_ref, w_ref, b_ref, o_ref):
    o_ref[...] = (
        jnp.dot(x_ref[...], w_ref[...], preferred_element_type=jnp.float32)
        + b_ref[...]
    )


def _fc(x, w, b):
    return pl.pallas_call(
        _fc_body,
        out_shape=jax.ShapeDtypeStruct((B, OUT), jnp.float32),
    )(x, w, b.reshape(1, OUT))


def kernel(x, edge_index, W0a, b0a, W0b, b0b, W1a, b1a, W1b, b1b, W2a, b2a,
           W2b, b2b, W3a, b3a, W3b, b3b, W4a, b4a, W4b, b4b, Wfc, bfc):
    src = edge_index[0]
    dst = edge_index[1]
    layers = [
        (W0a, b0a, W0b, b0b),
        (W1a, b1a, W1b, b1b),
        (W2a, b2a, W2b, b2b),
        (W3a, b3a, W3b, b3b),
        (W4a, b4a, W4b, b4b),
    ]
    h = x
    sc_aggregate = _make_sc_aggregate()
    srcr = src.reshape(NW, NHALF, NCH, CH)
    dstr = dst.reshape(NW, NHALF, NCH, CH)
    zeros = jnp.zeros((N, D), jnp.float32)
    for (Wa, ba, Wb, bb) in layers:
        parts = sc_aggregate(h, zeros, srcr, dstr)
        h = _mlp(parts[0], parts[1], Wa, ba, Wb, bb)
    return _fc(h.reshape(B, -1), Wfc, bfc)


# speedup vs baseline: 1.1297x; 1.0514x over previous
import functools

import jax
import jax.numpy as jnp
from jax import lax
from jax.experimental import pallas as pl
from jax.experimental.pallas import tpu as pltpu
from jax.experimental.pallas import tpu_sc as plsc

N = 10000
E = 320000
D = 128
OUT = 10
B = 2000

NC = 2
NS = 16
NW = NC * NS
EPW = E // NW
CH = 125
NCHUNK = EPW // CH
NHALF = 5
NCH = NCHUNK // NHALF
RPT = 624
RTAIL = N - NS * RPT

@functools.cache
def _make_sc_aggregate():
    mesh = plsc.VectorSubcoreMesh(core_axis_name="c", subcore_axis_name="s")

    @functools.partial(
        pl.kernel,
        mesh=mesh,
        out_type=(
            jax.ShapeDtypeStruct((N, D), jnp.float32),
            jax.ShapeDtypeStruct((N, D), jnp.float32),
        ),
        scratch_types=[
            pltpu.VMEM((2, NCH, CH), jnp.int32),
            pltpu.VMEM((2, NCH, CH), jnp.int32),
            pltpu.VMEM((2, CH, D), jnp.float32),
            pltpu.VMEM_SHARED((N, D), jnp.float32),
            pltpu.SemaphoreType.DMA,
            pltpu.SemaphoreType.DMA,
            pltpu.SemaphoreType.DMA,
            pltpu.SemaphoreType.DMA,
        ],
    )
    def _sc_aggregate(h_hbm, z_hbm, srcr_hbm, dstr_hbm, out0_hbm, out1_hbm,
                      sidx, didx, rows, acc, gsem_a, gsem_b, isem, csem):
        c = lax.axis_index("c")
        s = lax.axis_index("s")
        wid = s * NC + c
        r0 = s * RPT
        init = (h_hbm, z_hbm)
        for cc in range(NC):

            @pl.when(c == cc)
            def _(cc=cc):
                pltpu.async_copy(
                    init[cc].at[pl.ds(r0, RPT)], acc.at[pl.ds(r0, RPT)], csem
                )

                @pl.when(s == NS - 1)
                def _():
                    pltpu.async_copy(
                        init[cc].at[pl.ds(NS * RPT, RTAIL)],
                        acc.at[pl.ds(NS * RPT, RTAIL)],
                        csem,
                    )

        sems = (gsem_a, gsem_b)

        def stage(hf, qb, do_wait):
            cp_s = pltpu.async_copy(srcr_hbm.at[wid, hf], sidx.at[qb], isem)
            cp_d = pltpu.async_copy(dstr_hbm.at[wid, hf], didx.at[qb], isem)
            if do_wait:
                cp_s.wait()
                cp_d.wait()

        def fire(qb, j, bank):
            pltpu.async_copy(h_hbm.at[sidx.at[qb, j]], rows.at[bank], sems[bank])

        def drain_scatter(qb, j, bank):
            pltpu.make_async_copy(
                h_hbm.at[sidx.at[0, 0]], rows.at[bank], sems[bank]
            ).wait()
            pltpu.sync_copy(rows.at[bank], acc.at[didx.at[qb, j]], add=True)

        stage(0, 0, True)
        fire(0, 0, 0)

        pltpu.make_async_copy(
            h_hbm.at[pl.ds(r0, RPT)], acc.at[pl.ds(r0, RPT)], csem
        ).wait()

        @pl.when(s == NS - 1)
        def _():
            pltpu.make_async_copy(
                h_hbm.at[pl.ds(NS * RPT, RTAIL)], acc.at[pl.ds(NS * RPT, RTAIL)],
                csem,
            ).wait()

        plsc.subcore_barrier()

        for hf in range(NHALF):
            qb = hf % 2
            if hf + 1 < NHALF:
                stage(hf + 1, 1 - qb, False)

            def body(i, carry, qb=qb):
                j0 = i * 2
                fire(qb, j0 + 1, 1)
                drain_scatter(qb, j0, 0)

                @pl.when(j0 + 2 < NCH)
                def _():
                    fire(qb, j0 + 2, 0)

                drain_scatter(qb, j0 + 1, 1)
                return carry

            lax.fori_loop(0, NCH // 2, body, 0)

            if hf + 1 < NHALF:
                pltpu.make_async_copy(
                    srcr_hbm.at[wid, hf + 1], sidx.at[1 - qb], isem
                ).wait()
                pltpu.make_async_copy(
                    dstr_hbm.at[wid, hf + 1], didx.at[1 - qb], isem
                ).wait()
                fire(1 - qb, 0, 0)
        plsc.subcore_barrier()
        outs = (out0_hbm, out1_hbm)
        for cc in range(NC):

            @pl.when(c == cc)
            def _(cc=cc):
                pltpu.sync_copy(
                    acc.at[pl.ds(r0, RPT)], outs[cc].at[pl.ds(r0, RPT)]
                )

                @pl.when(s == NS - 1)
                def _():
                    pltpu.sync_copy(
                        acc.at[pl.ds(NS * RPT, RTAIL)],
                        outs[cc].at[pl.ds(NS * RPT, RTAIL)],
                    )

    return _sc_aggregate


def _mlp_body(p0_ref, p1_ref, wa_ref, ba_ref, wb_ref, bb_ref, o_ref):
    m = p0_ref[...] + p1_ref[...]
    t = jnp.maximum(
        jnp.dot(m, wa_ref[...], preferred_element_type=jnp.float32) + ba_ref[...],
        0.0,
    )
    o_ref[...] = jnp.maximum(
        jnp.dot(t, wb_ref[...], preferred_element_type=jnp.float32) + bb_ref[...],
        0.0,
    )


_ROWBLK = 2000


def _mlp(p0, p1, wa, ba, wb, bb):
    return pl.pallas_call(
        _mlp_body,
        grid=(N // _ROWBLK,),
        in_specs=[
            pl.BlockSpec((_ROWBLK, D), lambda i: (i, 0)),
            pl.BlockSpec((_ROWBLK, D), lambda i: (i, 0)),
            pl.BlockSpec((D, D), lambda i: (0, 0)),
            pl.BlockSpec((1, D), lambda i: (0, 0)),
            pl.BlockSpec((D, D), lambda i: (0, 0)),
            pl.BlockSpec((1, D), lambda i: (0, 0)),
        ],
        out_specs=pl.BlockSpec((_ROWBLK, D), lambda i: (i, 0)),
        out_shape=jax.ShapeDtypeStruct((N, D), jnp.float32),
    )(p0, p1, wa, ba.reshape(1, D), wb, bb.reshape(1, D))


def _fc_body(x_ref, w_ref, b_ref, o_ref):
    o_ref[...] = (
        jnp.dot(x_ref[...], w_ref[...], preferred_element_type=jnp.float32)
        + b_ref[...]
    )


def _fc(x, w, b):
    return pl.pallas_call(
        _fc_body,
        out_shape=jax.ShapeDtypeStruct((B, OUT), jnp.float32),
    )(x, w, b.reshape(1, OUT))


def kernel(x, edge_index, W0a, b0a, W0b, b0b, W1a, b1a, W1b, b1b, W2a, b2a,
           W2b, b2b, W3a, b3a, W3b, b3b, W4a, b4a, W4b, b4b, Wfc, bfc):
    src = edge_index[0]
    dst = edge_index[1]
    layers = [
        (W0a, b0a, W0b, b0b),
        (W1a, b1a, W1b, b1b),
        (W2a, b2a, W2b, b2b),
        (W3a, b3a, W3b, b3b),
        (W4a, b4a, W4b, b4b),
    ]
    h = x
    sc_aggregate = _make_sc_aggregate()
    srcr = src.reshape(NW, NHALF, NCH, CH)
    dstr = dst.reshape(NW, NHALF, NCH, CH)
    zeros = jnp.zeros((N, D), jnp.float32)
    for (Wa, ba, Wb, bb) in layers:
        p0, p1 = sc_aggregate(h, zeros, srcr, dstr)
        h = _mlp(p0, p1, Wa, ba, Wb, bb)
    return _fc(h.reshape(B, -1), Wfc, bfc)


# speedup vs baseline: 1.1461x; 1.0145x over previous
import functools

import jax
import jax.numpy as jnp
from jax import lax
from jax.experimental import pallas as pl
from jax.experimental.pallas import tpu as pltpu
from jax.experimental.pallas import tpu_sc as plsc

N = 10000
E = 320000
D = 128
OUT = 10
B = 2000

NC = 2
NS = 16
NW = NC * NS
EPW = E // NW
CH = 125
NCHUNK = EPW // CH
NHALF = 5
NCH = NCHUNK // NHALF
RPT = 624
RTAIL = N - NS * RPT

@functools.cache
def _make_sc_aggregate():
    mesh = plsc.VectorSubcoreMesh(core_axis_name="c", subcore_axis_name="s")

    @functools.partial(
        pl.kernel,
        mesh=mesh,
        out_type=(
            jax.ShapeDtypeStruct((N, D), jnp.float32),
            jax.ShapeDtypeStruct((N, D), jnp.float32),
        ),
        scratch_types=[
            pltpu.VMEM((2, NCH, CH), jnp.int32),
            pltpu.VMEM((2, NCH, CH), jnp.int32),
            pltpu.VMEM((2, CH, D), jnp.float32),
            pltpu.VMEM_SHARED((N, D), jnp.float32),
            pltpu.SemaphoreType.DMA,
            pltpu.SemaphoreType.DMA,
            pltpu.SemaphoreType.DMA,
            pltpu.SemaphoreType.DMA,
        ],
    )
    def _sc_aggregate(h_hbm, z_hbm, srcr_hbm, dstr_hbm, out0_hbm, out1_hbm,
                      sidx, didx, rows, acc, gsem_a, gsem_b, isem, csem):
        c = lax.axis_index("c")
        s = lax.axis_index("s")
        wid = s * NC + c
        r0 = s * RPT
        init = (h_hbm, z_hbm)
        for cc in range(NC):

            @pl.when(c == cc)
            def _(cc=cc):
                pltpu.async_copy(
                    init[cc].at[pl.ds(r0, RPT)], acc.at[pl.ds(r0, RPT)], csem
                )

                @pl.when(s == NS - 1)
                def _():
                    pltpu.async_copy(
                        init[cc].at[pl.ds(NS * RPT, RTAIL)],
                        acc.at[pl.ds(NS * RPT, RTAIL)],
                        csem,
                    )

        sems = (gsem_a, gsem_b)

        def stage(hf, qb, do_wait):
            cp_s = pltpu.async_copy(srcr_hbm.at[wid, hf], sidx.at[qb], isem)
            cp_d = pltpu.async_copy(dstr_hbm.at[wid, hf], didx.at[qb], isem)
            if do_wait:
                cp_s.wait()
                cp_d.wait()

        def fire(qb, j, bank):
            pltpu.async_copy(h_hbm.at[sidx.at[qb, j]], rows.at[bank], sems[bank])

        def drain_scatter(qb, j, bank):
            pltpu.make_async_copy(
                h_hbm.at[sidx.at[0, 0]], rows.at[bank], sems[bank]
            ).wait()
            pltpu.sync_copy(rows.at[bank], acc.at[didx.at[qb, j]], add=True)

        stage(0, 0, True)
        fire(0, 0, 0)

        pltpu.make_async_copy(
            h_hbm.at[pl.ds(r0, RPT)], acc.at[pl.ds(r0, RPT)], csem
        ).wait()

        @pl.when(s == NS - 1)
        def _():
            pltpu.make_async_copy(
                h_hbm.at[pl.ds(NS * RPT, RTAIL)], acc.at[pl.ds(NS * RPT, RTAIL)],
                csem,
            ).wait()

        plsc.subcore_barrier()

        for hf in range(NHALF):
            qb = hf % 2
            if hf + 1 < NHALF:
                stage(hf + 1, 1 - qb, False)

            def body(i, carry, qb=qb):
                j0 = i * 2
                fire(qb, j0 + 1, 1)
                drain_scatter(qb, j0, 0)

                @pl.when(j0 + 2 < NCH)
                def _():
                    fire(qb, j0 + 2, 0)

                drain_scatter(qb, j0 + 1, 1)
                return carry

            lax.fori_loop(0, NCH // 2, body, 0)

            if hf + 1 < NHALF:
                pltpu.make_async_copy(
                    srcr_hbm.at[wid, hf + 1], sidx.at[1 - qb], isem
                ).wait()
                pltpu.make_async_copy(
                    dstr_hbm.at[wid, hf + 1], didx.at[1 - qb], isem
                ).wait()
                fire(1 - qb, 0, 0)
        plsc.subcore_barrier()
        outs = (out0_hbm, out1_hbm)
        for cc in range(NC):

            @pl.when(c == cc)
            def _(cc=cc):
                pltpu.sync_copy(
                    acc.at[pl.ds(r0, RPT)], outs[cc].at[pl.ds(r0, RPT)]
                )

                @pl.when(s == NS - 1)
                def _():
                    pltpu.sync_copy(
                        acc.at[pl.ds(NS * RPT, RTAIL)],
                        outs[cc].at[pl.ds(NS * RPT, RTAIL)],
                    )

    return _sc_aggregate


def _mlp_body(p0_ref, p1_ref, wa_ref, ba_ref, wb_ref, bb_ref, o_ref):
    m = p0_ref[...] + p1_ref[...]
    t = jnp.maximum(
        jnp.dot(m, wa_ref[...], preferred_element_type=jnp.float32) + ba_ref[...],
        0.0,
    )
    o_ref[...] = jnp.maximum(
        jnp.dot(t, wb_ref[...], preferred_element_type=jnp.float32) + bb_ref[...],
        0.0,
    )


_ROWBLK = 2000


def _mlp(p0, p1, wa, ba, wb, bb):
    return pl.pallas_call(
        _mlp_body,
        grid=(N // _ROWBLK,),
        in_specs=[
            pl.BlockSpec((_ROWBLK, D), lambda i: (i, 0)),
            pl.BlockSpec((_ROWBLK, D), lambda i: (i, 0)),
            pl.BlockSpec((D, D), lambda i: (0, 0)),
            pl.BlockSpec((1, D), lambda i: (0, 0)),
            pl.BlockSpec((D, D), lambda i: (0, 0)),
            pl.BlockSpec((1, D), lambda i: (0, 0)),
        ],
        out_specs=pl.BlockSpec((_ROWBLK, D), lambda i: (i, 0)),
        out_shape=jax.ShapeDtypeStruct((N, D), jnp.float32),
    )(p0, p1, wa, ba.reshape(1, D), wb, bb.reshape(1, D))


def _mlp_fc_body(p0_ref, p1_ref, wa_ref, ba_ref, wb_ref, bb_ref, wfc_ref,
                 bfc_ref, o_ref):
    m = p0_ref[...] + p1_ref[...]
    t = jnp.maximum(
        jnp.dot(m, wa_ref[...], preferred_element_type=jnp.float32) + ba_ref[...],
        0.0,
    )
    u = jnp.maximum(
        jnp.dot(t, wb_ref[...], preferred_element_type=jnp.float32) + bb_ref[...],
        0.0,
    )
    xr = u.reshape(_ROWBLK // 5, 5 * D)
    o_ref[...] = (
        jnp.dot(xr, wfc_ref[...], preferred_element_type=jnp.float32)
        + bfc_ref[...]
    )


def _mlp_fc(p0, p1, wa, ba, wb, bb, wfc, bfc):
    return pl.pallas_call(
        _mlp_fc_body,
        grid=(N // _ROWBLK,),
        in_specs=[
            pl.BlockSpec((_ROWBLK, D), lambda i: (i, 0)),
            pl.BlockSpec((_ROWBLK, D), lambda i: (i, 0)),
            pl.BlockSpec((D, D), lambda i: (0, 0)),
            pl.BlockSpec((1, D), lambda i: (0, 0)),
            pl.BlockSpec((D, D), lambda i: (0, 0)),
            pl.BlockSpec((1, D), lambda i: (0, 0)),
            pl.BlockSpec((5 * D, OUT), lambda i: (0, 0)),
            pl.BlockSpec((1, OUT), lambda i: (0, 0)),
        ],
        out_specs=pl.BlockSpec((_ROWBLK // 5, OUT), lambda i: (i, 0)),
        out_shape=jax.ShapeDtypeStruct((B, OUT), jnp.float32),
    )(p0, p1, wa, ba.reshape(1, D), wb, bb.reshape(1, D), wfc,
      bfc.reshape(1, OUT))


def _fc_body(x_ref, w_ref, b_ref, o_ref):
    o_ref[...] = (
        jnp.dot(x_ref[...], w_ref[...], preferred_element_type=jnp.float32)
        + b_ref[...]
    )


def _fc(x, w, b):
    return pl.pallas_call(
        _fc_body,
        out_shape=jax.ShapeDtypeStruct((B, OUT), jnp.float32),
    )(x, w, b.reshape(1, OUT))


def kernel(x, edge_index, W0a, b0a, W0b, b0b, W1a, b1a, W1b, b1b, W2a, b2a,
           W2b, b2b, W3a, b3a, W3b, b3b, W4a, b4a, W4b, b4b, Wfc, bfc):
    src = edge_index[0]
    dst = edge_index[1]
    layers = [
        (W0a, b0a, W0b, b0b),
        (W1a, b1a, W1b, b1b),
        (W2a, b2a, W2b, b2b),
        (W3a, b3a, W3b, b3b),
        (W4a, b4a, W4b, b4b),
    ]
    h = x
    sc_aggregate = _make_sc_aggregate()
    srcr = src.reshape(NW, NHALF, NCH, CH)
    dstr = dst.reshape(NW, NHALF, NCH, CH)
    zeros = jnp.zeros((N, D), jnp.float32)
    for (Wa, ba, Wb, bb) in layers[:-1]:
        p0, p1 = sc_aggregate(h, zeros, srcr, dstr)
        h = _mlp(p0, p1, Wa, ba, Wb, bb)
    (Wa, ba, Wb, bb) = layers[-1]
    p0, p1 = sc_aggregate(h, zeros, srcr, dstr)
    return _mlp_fc(p0, p1, Wa, ba, Wb, bb, Wfc, bfc)
